# baseline (speedup 1.0000x reference)
import functools
import math

import jax
import jax.numpy as jnp
from jax.experimental import pallas as pl
from jax.experimental.pallas import tpu as pltpu

_SIM = 100
_GROUP = 4


def _group_kernel(a_ref, v_ref, mb_ref, wup_ref, bup_ref, wq_ref, bq_ref,
                  wk_ref, bk_ref, wo_ref, bo_ref, o_ref, *, n, c, sim, group):
    f32 = jnp.float32
    i32 = jnp.int32

    s_row = jax.nn.sigmoid(
        jnp.dot(mb_ref[...], wup_ref[...], preferred_element_type=f32)
        + bup_ref[...])
    s0 = s_row[0, 0]
    s_pad = jnp.concatenate(
        [s_row[:, 1:], jnp.zeros((1, n - (s_row.shape[1] - 1)), f32)], axis=1)
    s_tab = jnp.broadcast_to(s_pad, (n, n))

    jr = jax.lax.broadcasted_iota(i32, (n, n), 0)
    jc = jax.lax.broadcasted_iota(i32, (n, n), 1)
    tri = (jc < jr).astype(f32)

    int_min = jnp.iinfo(jnp.int32).min
    int_max = jnp.iinfo(jnp.int32).max
    inv_sqrt_d = f32(1.0 / math.sqrt(c))

    for g in range(group):
        A = a_ref[g]
        V = v_ref[g]

        AT = jnp.transpose(A)
        xbits = jax.lax.bitcast_convert_type(AT, i32)
        key = jnp.where(xbits < 0, xbits ^ i32(0x7FFFFFFF), xbits)

        lo = jnp.full((1, n), int_min, i32)
        hi = jnp.full((1, n), int_max, i32)
        for _ in range(32):
            mid = (lo >> 1) + (hi >> 1) + (lo & hi & 1)
            cnt = jnp.sum((key > mid).astype(i32), axis=0, keepdims=True)
            pred = cnt >= sim
            lo = jnp.where(pred, mid, lo)
            hi = jnp.where(pred, hi, mid)
        T = hi

        gt = key > T
        eq = key == T
        eq_f = eq.astype(f32)
        m = jnp.sum(gt.astype(i32), axis=0, keepdims=True)
        r_f = (sim - m).astype(f32)

        tiepos = jnp.dot(tri, eq_f, preferred_element_type=f32)
        selT = gt | (eq & (tiepos < r_f))
        sel_f = selT.astype(f32)
        posT = jnp.dot(tri, sel_f, preferred_element_type=f32)

        scaleT = jnp.take_along_axis(s_tab, posT.astype(i32), axis=1)

        Q = (s0 * jnp.dot(V, wq_ref[...], preferred_element_type=f32)
             + bq_ref[...])
        K = jnp.dot(V, wk_ref[...], preferred_element_type=f32)
        QKT = jax.lax.dot_general(K, Q, (((1,), (1,)), ((), ())),
                                  preferred_element_type=f32)
        qbkT = jax.lax.dot_general(bk_ref[...], Q, (((1,), (1,)), ((), ())),
                                   preferred_element_type=f32)
        scores = jnp.where(selT, (scaleT * QKT + qbkT) * inv_sqrt_d,
                           f32(-1e30))
        mx = jnp.max(scores, axis=0, keepdims=True)
        e = jnp.exp(scores - mx)
        w = e / jnp.sum(e, axis=0, keepdims=True)
        ws = w * scaleT
        out = jax.lax.dot_general(ws, V, (((0,), (0,)), ((), ())),
                                  preferred_element_type=f32)
        out = (jnp.dot(out, wo_ref[...], preferred_element_type=f32)
               + bo_ref[...])
        o_ref[g] = out


def kernel(attn, v, W_sub, b_sub, W_up, b_up, mb, Wq, bq, Wk, bk, Wo, bo):
    b, N, Nw, _, n, _ = attn.shape
    c = v.shape[-1]
    T = b * N * Nw
    A = attn.reshape(T, n, n)
    V = v.reshape(T, n, c)
    gch = W_up.shape[0]
    lch = W_up.shape[1]
    grp = math.gcd(_GROUP, T)

    mb_row = mb.reshape(1, lch)
    WupT = W_up.T
    bup_row = b_up.reshape(1, gch)
    WqT, WkT, WoT = Wq.T, Wk.T, Wo.T
    bq_row = bq.reshape(1, c)
    bk_row = bk.reshape(1, c)
    bo_row = bo.reshape(1, c)

    tile = pl.BlockSpec((grp, n, n), lambda i: (i, 0, 0))
    vtile = pl.BlockSpec((grp, n, c), lambda i: (i, 0, 0))

    def _const(shape):
        nd = len(shape)
        return pl.BlockSpec(shape, lambda i: (0,) * nd)

    out = pl.pallas_call(
        functools.partial(_group_kernel, n=n, c=c, sim=_SIM, group=grp),
        grid=(T // grp,),
        in_specs=[
            tile, vtile,
            _const((1, lch)), _const((lch, gch)), _const((1, gch)),
            _const((c, c)), _const((1, c)),
            _const((c, c)), _const((1, c)),
            _const((c, c)), _const((1, c)),
        ],
        out_specs=vtile,
        out_shape=jax.ShapeDtypeStruct((T, n, c), jnp.float32),
        compiler_params=pltpu.CompilerParams(
            dimension_semantics=("parallel",)),
    )(A, V, mb_row, WupT, bup_row, WqT, bq_row, WkT, bk_row, WoT, bo_row)
    return out.reshape(b, N, Nw, n, c)


# speedup vs baseline: 1.0603x; 1.0603x over previous
import functools
import math

import jax
import jax.numpy as jnp
from jax.experimental import pallas as pl
from jax.experimental.pallas import tpu as pltpu

_SIM = 100
_GROUP = 16


def _group_kernel(a_ref, v_ref, mb_ref, wup_ref, bup_ref, wq_ref, bq_ref,
                  wk_ref, bk_ref, wo_ref, bo_ref, o_ref, *, n, c, sim, group):
    f32 = jnp.float32
    i32 = jnp.int32

    s_row = jax.nn.sigmoid(
        jnp.dot(mb_ref[...], wup_ref[...], preferred_element_type=f32)
        + bup_ref[...])
    s0 = s_row[0, 0]
    s_pad = jnp.concatenate(
        [s_row[:, 1:], jnp.zeros((1, n - (s_row.shape[1] - 1)), f32)], axis=1)
    s_tab = jnp.broadcast_to(s_pad, (n, n))

    jr = jax.lax.broadcasted_iota(i32, (n, n), 0)
    jc = jax.lax.broadcasted_iota(i32, (n, n), 1)
    tri = (jc < jr).astype(f32)

    int_min = jnp.iinfo(jnp.int32).min
    int_max = jnp.iinfo(jnp.int32).max
    inv_sqrt_d = f32(1.0 / math.sqrt(c))

    for g in range(group):
        A = a_ref[g]
        V = v_ref[g]

        AT = jnp.transpose(A)
        xbits = jax.lax.bitcast_convert_type(AT, i32)
        key = jnp.where(xbits < 0, xbits ^ i32(0x7FFFFFFF), xbits)

        lo = jnp.full((1, n), int_min, i32)
        hi = jnp.full((1, n), int_max, i32)
        for _ in range(32):
            mid = (lo >> 1) + (hi >> 1) + (lo & hi & 1)
            cnt = jnp.sum((key > mid).astype(i32), axis=0, keepdims=True)
            pred = cnt >= sim
            lo = jnp.where(pred, mid, lo)
            hi = jnp.where(pred, hi, mid)
        T = hi

        gt = key > T
        eq = key == T
        eq_f = eq.astype(f32)
        m = jnp.sum(gt.astype(i32), axis=0, keepdims=True)
        r_f = (sim - m).astype(f32)

        tiepos = jnp.dot(tri, eq_f, preferred_element_type=f32)
        selT = gt | (eq & (tiepos < r_f))
        sel_f = selT.astype(f32)
        posT = jnp.dot(tri, sel_f, preferred_element_type=f32)

        scaleT = jnp.take_along_axis(s_tab, posT.astype(i32), axis=1)

        Q = (s0 * jnp.dot(V, wq_ref[...], preferred_element_type=f32)
             + bq_ref[...])
        K = jnp.dot(V, wk_ref[...], preferred_element_type=f32)
        QKT = jax.lax.dot_general(K, Q, (((1,), (1,)), ((), ())),
                                  preferred_element_type=f32)
        qbkT = jax.lax.dot_general(bk_ref[...], Q, (((1,), (1,)), ((), ())),
                                   preferred_element_type=f32)
        scores = jnp.where(selT, (scaleT * QKT + qbkT) * inv_sqrt_d,
                           f32(-1e30))
        mx = jnp.max(scores, axis=0, keepdims=True)
        e = jnp.exp(scores - mx)
        w = e / jnp.sum(e, axis=0, keepdims=True)
        ws = w * scaleT
        out = jax.lax.dot_general(ws, V, (((0,), (0,)), ((), ())),
                                  preferred_element_type=f32)
        out = (jnp.dot(out, wo_ref[...], preferred_element_type=f32)
               + bo_ref[...])
        o_ref[g] = out


def kernel(attn, v, W_sub, b_sub, W_up, b_up, mb, Wq, bq, Wk, bk, Wo, bo):
    b, N, Nw, _, n, _ = attn.shape
    c = v.shape[-1]
    T = b * N * Nw
    A = attn.reshape(T, n, n)
    V = v.reshape(T, n, c)
    gch = W_up.shape[0]
    lch = W_up.shape[1]
    grp = math.gcd(_GROUP, T)

    mb_row = mb.reshape(1, lch)
    WupT = W_up.T
    bup_row = b_up.reshape(1, gch)
    WqT, WkT, WoT = Wq.T, Wk.T, Wo.T
    bq_row = bq.reshape(1, c)
    bk_row = bk.reshape(1, c)
    bo_row = bo.reshape(1, c)

    tile = pl.BlockSpec((grp, n, n), lambda i: (i, 0, 0))
    vtile = pl.BlockSpec((grp, n, c), lambda i: (i, 0, 0))

    def _const(shape):
        nd = len(shape)
        return pl.BlockSpec(shape, lambda i: (0,) * nd)

    out = pl.pallas_call(
        functools.partial(_group_kernel, n=n, c=c, sim=_SIM, group=grp),
        grid=(T // grp,),
        in_specs=[
            tile, vtile,
            _const((1, lch)), _const((lch, gch)), _const((1, gch)),
            _const((c, c)), _const((1, c)),
            _const((c, c)), _const((1, c)),
            _const((c, c)), _const((1, c)),
        ],
        out_specs=vtile,
        out_shape=jax.ShapeDtypeStruct((T, n, c), jnp.float32),
        compiler_params=pltpu.CompilerParams(
            dimension_semantics=("parallel",)),
    )(A, V, mb_row, WupT, bup_row, WqT, bq_row, WkT, bk_row, WoT, bo_row)
    return out.reshape(b, N, Nw, n, c)


# speedup vs baseline: 1.1910x; 1.1233x over previous
import functools
import math

import jax
import jax.numpy as jnp
from jax.experimental import pallas as pl
from jax.experimental.pallas import tpu as pltpu

_SIM = 100
_GROUP = 16


def _group_kernel(a_ref, v_ref, mb_ref, wup_ref, bup_ref, wq_ref, bq_ref,
                  wk_ref, bk_ref, wo_ref, bo_ref, o_ref, *, n, c, sim, group):
    f32 = jnp.float32
    i32 = jnp.int32

    s_row = jax.nn.sigmoid(
        jax.lax.dot_general(mb_ref[...], wup_ref[...], (((1,), (1,)), ((), ())),
                            preferred_element_type=f32)
        + bup_ref[...])
    s0 = s_row[0, 0]
    s_pad = jnp.concatenate(
        [s_row[:, 1:], jnp.zeros((1, n - (s_row.shape[1] - 1)), f32)], axis=1)
    s_tab = jnp.broadcast_to(s_pad, (n, n))

    jr = jax.lax.broadcasted_iota(i32, (n, n), 0)
    jc = jax.lax.broadcasted_iota(i32, (n, n), 1)
    tri = (jc < jr).astype(f32)

    int_min = jnp.iinfo(jnp.int32).min
    int_max = jnp.iinfo(jnp.int32).max
    inv_sqrt_d = f32(1.0 / math.sqrt(c))

    for g in range(group):
        A = a_ref[g]
        V = v_ref[g]

        AT = jnp.transpose(A)
        xbits = jax.lax.bitcast_convert_type(AT, i32)
        key = jnp.where(xbits < 0, xbits ^ i32(0x7FFFFFFF), xbits)

        lo = jnp.full((1, n), int_min, i32)
        hi = jnp.full((1, n), int_max, i32)
        for _ in range(32):
            mid = (lo >> 1) + (hi >> 1) + (lo & hi & 1)
            cnt = jnp.sum((key > mid).astype(i32), axis=0, keepdims=True)
            pred = cnt >= sim
            lo = jnp.where(pred, mid, lo)
            hi = jnp.where(pred, hi, mid)
        T = hi

        gt = key > T
        eq = key == T
        eq_f = eq.astype(f32)
        m = jnp.sum(gt.astype(i32), axis=0, keepdims=True)
        r_f = (sim - m).astype(f32)

        tiepos = jnp.dot(tri, eq_f, preferred_element_type=f32)
        selT = gt | (eq & (tiepos < r_f))
        sel_f = selT.astype(f32)
        posT = jnp.dot(tri, sel_f, preferred_element_type=f32)

        scaleT = jnp.take_along_axis(s_tab, posT.astype(i32), axis=1)

        Q = (s0 * jax.lax.dot_general(V, wq_ref[...], (((1,), (1,)), ((), ())),
                                      preferred_element_type=f32)
             + bq_ref[...])
        K = jax.lax.dot_general(V, wk_ref[...], (((1,), (1,)), ((), ())),
                                preferred_element_type=f32)
        QKT = jax.lax.dot_general(K, Q, (((1,), (1,)), ((), ())),
                                  preferred_element_type=f32)
        qbkT = jax.lax.dot_general(bk_ref[...], Q, (((1,), (1,)), ((), ())),
                                   preferred_element_type=f32)
        scores = jnp.where(selT, (scaleT * QKT + qbkT) * inv_sqrt_d,
                           f32(-1e30))
        mx = jnp.max(scores, axis=0, keepdims=True)
        e = jnp.exp(scores - mx)
        w = e / jnp.sum(e, axis=0, keepdims=True)
        ws = w * scaleT
        out = jax.lax.dot_general(ws, V, (((0,), (0,)), ((), ())),
                                  preferred_element_type=f32)
        out = (jax.lax.dot_general(out, wo_ref[...], (((1,), (1,)), ((), ())),
                                   preferred_element_type=f32)
               + bo_ref[...])
        o_ref[g] = out


def kernel(attn, v, W_sub, b_sub, W_up, b_up, mb, Wq, bq, Wk, bk, Wo, bo):
    b, N, Nw, _, n, _ = attn.shape
    c = v.shape[-1]
    T = b * N * Nw
    A = attn.reshape(T, n, n)
    V = v.reshape(T, n, c)
    gch = W_up.shape[0]
    lch = W_up.shape[1]
    grp = math.gcd(_GROUP, T)

    mb_row = mb.reshape(1, lch)
    bup_row = b_up.reshape(1, gch)
    bq_row = bq.reshape(1, c)
    bk_row = bk.reshape(1, c)
    bo_row = bo.reshape(1, c)

    tile = pl.BlockSpec((grp, n, n), lambda i: (i, 0, 0))
    vtile = pl.BlockSpec((grp, n, c), lambda i: (i, 0, 0))

    def _const(shape):
        nd = len(shape)
        return pl.BlockSpec(shape, lambda i: (0,) * nd)

    out = pl.pallas_call(
        functools.partial(_group_kernel, n=n, c=c, sim=_SIM, group=grp),
        grid=(T // grp,),
        in_specs=[
            tile, vtile,
            _const((1, lch)), _const((gch, lch)), _const((1, gch)),
            _const((c, c)), _const((1, c)),
            _const((c, c)), _const((1, c)),
            _const((c, c)), _const((1, c)),
        ],
        out_specs=vtile,
        out_shape=jax.ShapeDtypeStruct((T, n, c), jnp.float32),
        compiler_params=pltpu.CompilerParams(
            dimension_semantics=("parallel",)),
    )(A, V, mb_row, W_up, bup_row, Wq, bq_row, Wk, bk_row, Wo, bo_row)
    return out.reshape(b, N, Nw, n, c)


# speedup vs baseline: 1.1928x; 1.0015x over previous
import functools
import math

import jax
import jax.numpy as jnp
from jax.experimental import pallas as pl
from jax.experimental.pallas import tpu as pltpu

_SIM = 100
_GROUP = 16


def _group_kernel(a_ref, v_ref, mb_ref, wup_ref, bup_ref, wq_ref, bq_ref,
                  wk_ref, bk_ref, wo_ref, bo_ref, o_ref, *, n, c, sim, group):
    f32 = jnp.float32
    i32 = jnp.int32

    s_row = jax.nn.sigmoid(
        jax.lax.dot_general(mb_ref[...], wup_ref[...], (((1,), (1,)), ((), ())),
                            preferred_element_type=f32)
        + bup_ref[...])
    s0 = s_row[0, 0]
    s_pad = jnp.concatenate(
        [s_row[:, 1:], jnp.zeros((1, n - (s_row.shape[1] - 1)), f32)], axis=1)
    s_tab = jnp.broadcast_to(s_pad, (n, n))

    jr = jax.lax.broadcasted_iota(i32, (n, n), 0)
    jc = jax.lax.broadcasted_iota(i32, (n, n), 1)
    tri = (jc < jr).astype(jnp.bfloat16)

    int_min = jnp.iinfo(jnp.int32).min
    int_max = jnp.iinfo(jnp.int32).max
    inv_sqrt_d = f32(1.0 / math.sqrt(c))

    for g in range(group):
        A = a_ref[g]
        V = v_ref[g]

        AT = jnp.transpose(A)
        xbits = jax.lax.bitcast_convert_type(AT, i32)
        key = jnp.where(xbits < 0, xbits ^ i32(0x7FFFFFFF), xbits)

        lo = jnp.full((1, n), int_min, i32)
        hi = jnp.full((1, n), int_max, i32)
        for _ in range(32):
            mid = (lo >> 1) + (hi >> 1) + (lo & hi & 1)
            cnt = jnp.sum((key > mid).astype(i32), axis=0, keepdims=True)
            pred = cnt >= sim
            lo = jnp.where(pred, mid, lo)
            hi = jnp.where(pred, hi, mid)
        T = hi

        gt = key > T
        eq = key == T
        m = jnp.sum(gt.astype(i32), axis=0, keepdims=True)
        r_f = (sim - m).astype(f32)

        tiepos = jnp.dot(tri, eq.astype(jnp.bfloat16),
                         preferred_element_type=f32)
        selT = gt | (eq & (tiepos < r_f))
        posT = jnp.dot(tri, selT.astype(jnp.bfloat16),
                       preferred_element_type=f32)

        scaleT = jnp.take_along_axis(s_tab, posT.astype(i32), axis=1)

        Q = (s0 * jax.lax.dot_general(V, wq_ref[...], (((1,), (1,)), ((), ())),
                                      preferred_element_type=f32)
             + bq_ref[...])
        K = jax.lax.dot_general(V, wk_ref[...], (((1,), (1,)), ((), ())),
                                preferred_element_type=f32)
        QKT = jax.lax.dot_general(K, Q, (((1,), (1,)), ((), ())),
                                  preferred_element_type=f32)
        qbkT = jax.lax.dot_general(bk_ref[...], Q, (((1,), (1,)), ((), ())),
                                   preferred_element_type=f32)
        scores = jnp.where(selT, (scaleT * QKT + qbkT) * inv_sqrt_d,
                           f32(-1e30))
        mx = jnp.max(scores, axis=0, keepdims=True)
        e = jnp.exp(scores - mx)
        w = e / jnp.sum(e, axis=0, keepdims=True)
        ws = w * scaleT
        out = jax.lax.dot_general(ws, V, (((0,), (0,)), ((), ())),
                                  preferred_element_type=f32)
        out = (jax.lax.dot_general(out, wo_ref[...], (((1,), (1,)), ((), ())),
                                   preferred_element_type=f32)
               + bo_ref[...])
        o_ref[g] = out


def kernel(attn, v, W_sub, b_sub, W_up, b_up, mb, Wq, bq, Wk, bk, Wo, bo):
    b, N, Nw, _, n, _ = attn.shape
    c = v.shape[-1]
    T = b * N * Nw
    A = attn.reshape(T, n, n)
    V = v.reshape(T, n, c)
    gch = W_up.shape[0]
    lch = W_up.shape[1]
    grp = math.gcd(_GROUP, T)

    mb_row = mb.reshape(1, lch)
    bup_row = b_up.reshape(1, gch)
    bq_row = bq.reshape(1, c)
    bk_row = bk.reshape(1, c)
    bo_row = bo.reshape(1, c)

    tile = pl.BlockSpec((grp, n, n), lambda i: (i, 0, 0))
    vtile = pl.BlockSpec((grp, n, c), lambda i: (i, 0, 0))

    def _const(shape):
        nd = len(shape)
        return pl.BlockSpec(shape, lambda i: (0,) * nd)

    out = pl.pallas_call(
        functools.partial(_group_kernel, n=n, c=c, sim=_SIM, group=grp),
        grid=(T // grp,),
        in_specs=[
            tile, vtile,
            _const((1, lch)), _const((gch, lch)), _const((1, gch)),
            _const((c, c)), _const((1, c)),
            _const((c, c)), _const((1, c)),
            _const((c, c)), _const((1, c)),
        ],
        out_specs=vtile,
        out_shape=jax.ShapeDtypeStruct((T, n, c), jnp.float32),
        compiler_params=pltpu.CompilerParams(
            dimension_semantics=("parallel",)),
    )(A, V, mb_row, W_up, bup_row, Wq, bq_row, Wk, bk_row, Wo, bo_row)
    return out.reshape(b, N, Nw, n, c)
